# baseline (speedup 1.0000x reference)
import functools

import jax
import jax.numpy as jnp
from jax import lax
from jax.experimental import pallas as pl
from jax.experimental.pallas import tpu as pltpu
from jax.experimental.pallas import tpu_sc as plsc

B = 16384
D_EMB = 64
N_IND_SLOTS = 20
IND_SZ = 16
FTYPE_SZ = 4
N_INDS = 1000
N_FTYPES = 16
AF_ROWS = IND_SZ + FTYPE_SZ

NC = 2
NS = 16
L = 16
NW = NC * NS
BPW = B // NW
HALF = BPW // 2
CHUNK_ROWS = 8
NGROUP = BPW // L


def _sc_body(funds_h, startups_h, ind_t_h, ftype_h, userT_h, itemT_h,
             itblT_h, ftblT_h,
             ui_out, af_out,
             vidx_u, vidx_i, ftidx_v, indv_v, itblT_v, ftblT_v,
             ring_u, ring_i, ui_stage, af_stage, sem):
  wid = lax.axis_index("s") * NC + lax.axis_index("c")
  base = wid * BPW

  pltpu.sync_copy(funds_h.at[pl.ds(base, BPW)], vidx_u)
  pltpu.sync_copy(startups_h.at[pl.ds(base, BPW)], vidx_i)
  pltpu.sync_copy(ftype_h.at[pl.ds(base, BPW)], ftidx_v)
  for j in range(N_IND_SLOTS):
    pltpu.sync_copy(ind_t_h.at[j, pl.ds(base, BPW)], indv_v.at[j])
  pltpu.sync_copy(itblT_h, itblT_v)
  pltpu.sync_copy(ftblT_h, ftblT_v)

  iota = lax.iota(jnp.int32, L)

  def group(g, carry):
    s = pl.multiple_of(g * L, L)
    idxs = [indv_v[j, pl.ds(s, L)] for j in range(N_IND_SLOTS)]
    cnt = jnp.full((L,), 0.0, jnp.float32)
    for j in range(N_IND_SLOTS):
      cnt = cnt + jnp.minimum(idxs[j], 1).astype(jnp.float32)
    inv = jnp.full((L,), 1.0, jnp.float32) / cnt
    for c in range(IND_SZ):
      colv = jnp.full((L,), c, jnp.int32)
      acc = plsc.load_gather(itblT_v, [colv, idxs[0]])
      for j in range(1, N_IND_SLOTS):
        acc = acc + plsc.load_gather(itblT_v, [colv, idxs[j]])
      af_stage[c, pl.ds(s, L)] = acc * inv
    ftv = ftidx_v[pl.ds(s, L)]
    for c in range(FTYPE_SZ):
      colv = jnp.full((L,), c, jnp.int32)
      af_stage[IND_SZ + c, pl.ds(s, L)] = plsc.load_gather(ftblT_v, [colv, ftv])
    return carry

  lax.fori_loop(0, NGROUP, group, 0)
  pltpu.sync_copy(af_stage, af_out.at[:, pl.ds(base, BPW)])

  def extract(ring, k, pvec, row_local, col_off):
    colv = pvec + (k * L)
    for c4 in range(D_EMB // L):
      fvec = iota + (c4 * L)
      vals = plsc.load_gather(ring, [fvec, colv])
      plsc.store_scatter(ui_stage,
                         [jnp.full((L,), row_local, jnp.int32), fvec + col_off],
                         vals)

  for half in range(2):
    hbase = half * HALF

    def chunk_body(ch, carry):
      row0 = ch * CHUNK_ROWS
      copies = []
      lanes_u, lanes_i = [], []
      for k in range(CHUNK_ROWS):
        rsplat = jnp.full((L,), hbase + row0 + k, jnp.int32)
        su = plsc.load_gather(vidx_u, [rsplat])
        si = plsc.load_gather(vidx_i, [rsplat])
        lanes_u.append(jnp.bitwise_and(su, L - 1))
        lanes_i.append(jnp.bitwise_and(si, L - 1))
        start_u = pl.multiple_of(jnp.max(jnp.bitwise_and(su, -L)), L)
        start_i = pl.multiple_of(jnp.max(jnp.bitwise_and(si, -L)), L)
        copies.append(pltpu.async_copy(
            userT_h.at[:, pl.ds(start_u, L)],
            ring_u.at[:, pl.ds(k * L, L)], sem))
        copies.append(pltpu.async_copy(
            itemT_h.at[:, pl.ds(start_i, L)],
            ring_i.at[:, pl.ds(k * L, L)], sem))
      for c in copies:
        c.wait()
      for k in range(CHUNK_ROWS):
        extract(ring_u, k, lanes_u[k], row0 + k, 0)
        extract(ring_i, k, lanes_i[k], row0 + k, D_EMB)
      return carry

    lax.fori_loop(0, HALF // CHUNK_ROWS, chunk_body, 0)
    pltpu.sync_copy(ui_stage, ui_out.at[pl.ds(base + hbase, HALF)])


def _sc_gather(funds, startups, industries_t, funding_type,
               userT, itemT, itblT, ftblT):
  mesh = plsc.VectorSubcoreMesh(
      core_axis_name="c", subcore_axis_name="s",
      num_cores=NC, num_subcores=NS)
  f32 = jnp.float32
  out_type = (
      jax.ShapeDtypeStruct((B, 2 * D_EMB), f32),
      jax.ShapeDtypeStruct((AF_ROWS, B), f32),
  )
  scratch = [
      pltpu.VMEM((BPW,), jnp.int32),
      pltpu.VMEM((BPW,), jnp.int32),
      pltpu.VMEM((BPW,), jnp.int32),
      pltpu.VMEM((N_IND_SLOTS, BPW), jnp.int32),
      pltpu.VMEM((IND_SZ, N_INDS), f32),
      pltpu.VMEM((FTYPE_SZ, N_FTYPES), f32),
      pltpu.VMEM((D_EMB, CHUNK_ROWS * L), f32),
      pltpu.VMEM((D_EMB, CHUNK_ROWS * L), f32),
      pltpu.VMEM((HALF, 2 * D_EMB), f32),
      pltpu.VMEM((AF_ROWS, BPW), f32),
      pltpu.SemaphoreType.DMA,
  ]
  run = pl.kernel(_sc_body, out_type=out_type, mesh=mesh,
                  scratch_types=scratch,
                  compiler_params=pltpu.CompilerParams(
                      use_tc_tiling_on_sc=False,
                      needs_layout_passes=False))
  return run(funds, startups, industries_t, funding_type,
             userT, itemT, itblT, ftblT)


BT = 2048


def _mlp_body(ui_ref, af_ref,
              w1u_ref, w1i_ref, w1af_ref, b1_ref,
              w2_ref, b2_ref, w3_ref, b3_ref, out_ref):
  dot = functools.partial(jnp.dot, preferred_element_type=jnp.float32)
  u = ui_ref[:, :D_EMB]
  it = ui_ref[:, D_EMB:]
  x = (dot(u, w1u_ref[:]) + dot(it, w1i_ref[:])
       + lax.dot_general(af_ref[:], w1af_ref[:], (((0,), (0,)), ((), ())),
                         preferred_element_type=jnp.float32)
       + b1_ref[:])
  h1 = jnp.maximum(x, 0.0)
  h2 = jnp.maximum(dot(h1, w2_ref[:]) + b2_ref[:], 0.0)
  out_ref[:] = jnp.maximum(dot(h2, w3_ref[:]) + b3_ref[:], 0.0)


def _mlp(ui, afT, W1u, W1i, W1af, b1, W2, b2, W3, b3):
  h1, h2, d_out = W2.shape[0], W3.shape[0], W3.shape[1]
  grid = (B // BT,)
  full = lambda shape: pl.BlockSpec(shape, lambda i: (0, 0))
  return pl.pallas_call(
      _mlp_body,
      grid=grid,
      in_specs=[
          pl.BlockSpec((BT, 2 * D_EMB), lambda i: (i, 0)),
          pl.BlockSpec((AF_ROWS, BT), lambda i: (0, i)),
          full(W1u.shape), full(W1i.shape), full(W1af.shape),
          full((1, h1)),
          full(W2.shape), full((1, h2)),
          full(W3.shape), full((1, d_out)),
      ],
      out_specs=pl.BlockSpec((BT, d_out), lambda i: (i, 0)),
      out_shape=jax.ShapeDtypeStruct((B, d_out), jnp.float32),
  )(ui, afT, W1u, W1i, W1af, b1.reshape(1, -1),
    W2, b2.reshape(1, -1), W3, b3.reshape(1, -1))


def kernel(funds, startups, industries, funding_type, user_table, item_table,
           ind_table, ftype_table, W1, b1, W2, b2, W3, b3):
  funds = funds.astype(jnp.int32)
  startups = startups.astype(jnp.int32)
  funding_type = funding_type.astype(jnp.int32)
  industries_t = industries.astype(jnp.int32).T
  itblT = ind_table.T.at[:, 0].set(0.0)
  ftblT = ftype_table.T
  ui, afT = _sc_gather(funds, startups, industries_t, funding_type,
                       user_table.T, item_table.T, itblT, ftblT)
  W1u = W1[:D_EMB]
  W1i = W1[D_EMB:2 * D_EMB]
  W1af = jnp.concatenate([W1[2 * D_EMB + FTYPE_SZ:], W1[2 * D_EMB:2 * D_EMB + FTYPE_SZ]], axis=0)
  return _mlp(ui, afT, W1u, W1i, W1af, b1, W2, b2, W3, b3)


# speedup vs baseline: 7.4354x; 7.4354x over previous
import functools

import jax
import jax.numpy as jnp
from jax import lax
from jax.experimental import pallas as pl
from jax.experimental.pallas import tpu as pltpu
from jax.experimental.pallas import tpu_sc as plsc

B = 16384
D_EMB = 64
N_IND_SLOTS = 20
IND_SZ = 16
FTYPE_SZ = 4
N_INDS = 1000
N_FTYPES = 16
AF_ROWS = IND_SZ + FTYPE_SZ

NC = 2
NS = 16
L = 16
NW = NC * NS
BPW = B // NW
CHUNK = 128
NCHUNK = BPW // CHUNK
NGROUP = BPW // L


def _sc_body(funds_h, startups_h, ind_t_h, ftype_h, user_h, item_h,
             itblT_h, ftblT_h,
             u_out, i_out, af_out,
             *refs):
  idxu_v = refs[0:NCHUNK]
  idxi_v = refs[NCHUNK:2 * NCHUNK]
  (ftidx_v, indv_v, itblT_v, ftblT_v,
   urows_v, irows_v, af_stage, sem) = refs[2 * NCHUNK:]
  wid = lax.axis_index("s") * NC + lax.axis_index("c")
  base = wid * BPW

  for k in range(NCHUNK):
    pltpu.sync_copy(funds_h.at[pl.ds(base + k * CHUNK, CHUNK)], idxu_v[k])
    pltpu.sync_copy(startups_h.at[pl.ds(base + k * CHUNK, CHUNK)], idxi_v[k])

  copies = []
  for k in range(NCHUNK):
    copies.append(pltpu.async_copy(
        user_h.at[idxu_v[k]], urows_v.at[pl.ds(k * CHUNK, CHUNK)], sem))
    copies.append(pltpu.async_copy(
        item_h.at[idxi_v[k]], irows_v.at[pl.ds(k * CHUNK, CHUNK)], sem))

  pltpu.sync_copy(itblT_h, itblT_v)
  pltpu.sync_copy(ftblT_h, ftblT_v)
  pltpu.sync_copy(ftype_h.at[pl.ds(base, BPW)], ftidx_v)
  for j in range(N_IND_SLOTS):
    pltpu.sync_copy(ind_t_h.at[j, pl.ds(base, BPW)], indv_v.at[j])

  iota = lax.iota(jnp.int32, L)

  def group(g, carry):
    s = pl.multiple_of(g * L, L)
    idxs = [indv_v[j, pl.ds(s, L)] for j in range(N_IND_SLOTS)]
    cnt = jnp.full((L,), 0.0, jnp.float32)
    for j in range(N_IND_SLOTS):
      cnt = cnt + jnp.minimum(idxs[j], 1).astype(jnp.float32)
    inv = jnp.full((L,), 1.0, jnp.float32) / cnt
    for c in range(IND_SZ):
      colv = jnp.full((L,), c, jnp.int32)
      acc = plsc.load_gather(itblT_v, [colv, idxs[0]])
      for j in range(1, N_IND_SLOTS):
        acc = acc + plsc.load_gather(itblT_v, [colv, idxs[j]])
      af_stage[c, pl.ds(s, L)] = acc * inv
    ftv = ftidx_v[pl.ds(s, L)]
    for c in range(FTYPE_SZ):
      colv = jnp.full((L,), c, jnp.int32)
      af_stage[IND_SZ + c, pl.ds(s, L)] = plsc.load_gather(ftblT_v, [colv, ftv])
    return carry

  lax.fori_loop(0, NGROUP, group, 0)
  pltpu.sync_copy(af_stage, af_out.at[:, pl.ds(base, BPW)])

  for c in copies:
    c.wait()

  pltpu.sync_copy(urows_v, u_out.at[pl.ds(base, BPW)])
  pltpu.sync_copy(irows_v, i_out.at[pl.ds(base, BPW)])


def _sc_gather(funds, startups, industries_t, funding_type,
               user_table, item_table, itblT, ftblT):
  mesh = plsc.VectorSubcoreMesh(
      core_axis_name="c", subcore_axis_name="s",
      num_cores=NC, num_subcores=NS)
  f32 = jnp.float32
  out_type = (
      jax.ShapeDtypeStruct((B, D_EMB), f32),
      jax.ShapeDtypeStruct((B, D_EMB), f32),
      jax.ShapeDtypeStruct((AF_ROWS, B), f32),
  )
  scratch = [
      *[pltpu.VMEM((CHUNK,), jnp.int32) for _ in range(NCHUNK)],
      *[pltpu.VMEM((CHUNK,), jnp.int32) for _ in range(NCHUNK)],
      pltpu.VMEM((BPW,), jnp.int32),
      pltpu.VMEM((N_IND_SLOTS, BPW), jnp.int32),
      pltpu.VMEM((IND_SZ, N_INDS), f32),
      pltpu.VMEM((FTYPE_SZ, N_FTYPES), f32),
      pltpu.VMEM((BPW, D_EMB), f32),
      pltpu.VMEM((BPW, D_EMB), f32),
      pltpu.VMEM((AF_ROWS, BPW), f32),
      pltpu.SemaphoreType.DMA,
  ]
  run = pl.kernel(_sc_body, out_type=out_type, mesh=mesh,
                  scratch_types=scratch,
                  compiler_params=pltpu.CompilerParams(
                      use_tc_tiling_on_sc=False,
                      needs_layout_passes=False))
  return run(funds, startups, industries_t, funding_type,
             user_table, item_table, itblT, ftblT)


BT = 2048


def _mlp_body(u_ref, i_ref, af_ref,
              w1u_ref, w1i_ref, w1af_ref, b1_ref,
              w2_ref, b2_ref, w3_ref, b3_ref, out_ref):
  dot = functools.partial(jnp.dot, preferred_element_type=jnp.float32)
  x = (dot(u_ref[:], w1u_ref[:]) + dot(i_ref[:], w1i_ref[:])
       + lax.dot_general(af_ref[:], w1af_ref[:], (((0,), (0,)), ((), ())),
                         preferred_element_type=jnp.float32)
       + b1_ref[:])
  h1 = jnp.maximum(x, 0.0)
  h2 = jnp.maximum(dot(h1, w2_ref[:]) + b2_ref[:], 0.0)
  out_ref[:] = jnp.maximum(dot(h2, w3_ref[:]) + b3_ref[:], 0.0)


def _mlp(u, it, afT, W1u, W1i, W1af, b1, W2, b2, W3, b3):
  h1, h2, d_out = W2.shape[0], W3.shape[0], W3.shape[1]
  grid = (B // BT,)
  full = lambda shape: pl.BlockSpec(shape, lambda i: (0, 0))
  return pl.pallas_call(
      _mlp_body,
      grid=grid,
      in_specs=[
          pl.BlockSpec((BT, D_EMB), lambda i: (i, 0)),
          pl.BlockSpec((BT, D_EMB), lambda i: (i, 0)),
          pl.BlockSpec((AF_ROWS, BT), lambda i: (0, i)),
          full(W1u.shape), full(W1i.shape), full(W1af.shape),
          full((1, h1)),
          full(W2.shape), full((1, h2)),
          full(W3.shape), full((1, d_out)),
      ],
      out_specs=pl.BlockSpec((BT, d_out), lambda i: (i, 0)),
      out_shape=jax.ShapeDtypeStruct((B, d_out), jnp.float32),
  )(u, it, afT, W1u, W1i, W1af, b1.reshape(1, -1),
    W2, b2.reshape(1, -1), W3, b3.reshape(1, -1))


def kernel(funds, startups, industries, funding_type, user_table, item_table,
           ind_table, ftype_table, W1, b1, W2, b2, W3, b3):
  funds = funds.astype(jnp.int32)
  startups = startups.astype(jnp.int32)
  funding_type = funding_type.astype(jnp.int32)
  industries_t = industries.astype(jnp.int32).T
  itblT = ind_table.T.at[:, 0].set(0.0)
  ftblT = ftype_table.T
  u, it, afT = _sc_gather(funds, startups, industries_t, funding_type,
                          user_table, item_table, itblT, ftblT)
  W1u = W1[:D_EMB]
  W1i = W1[D_EMB:2 * D_EMB]
  W1af = jnp.concatenate([W1[2 * D_EMB + FTYPE_SZ:], W1[2 * D_EMB:2 * D_EMB + FTYPE_SZ]], axis=0)
  return _mlp(u, it, afT, W1u, W1i, W1af, b1, W2, b2, W3, b3)
